# baseline (speedup 1.0000x reference)
import functools

import jax
import jax.numpy as jnp
import numpy as np
from jax import lax
from jax.experimental import pallas as pl
from jax.experimental.pallas import tpu as pltpu
from jax.experimental.pallas import tpu_sc as plsc

N_TOK = 32768
D_EMB = 512
S_DIM = 256

NC = 2
NS = 16
NW = NC * NS

P = 8
N_PIECE = N_TOK // P
B_PER_W = N_PIECE // NW
CHUNK = 64
NCHUNK = B_PER_W // CHUNK
GROUPS = D_EMB // 32

BM = 1024

def _sc_gather_piece(word_chunks, table):
    mesh = plsc.VectorSubcoreMesh(core_axis_name="c", subcore_axis_name="s")

    @functools.partial(
        pl.kernel,
        mesh=mesh,
        out_type=jax.ShapeDtypeStruct((N_PIECE, D_EMB // 2), jnp.int32),
        scratch_types=[
            pltpu.VMEM((NCHUNK, CHUNK), jnp.int32),
            pltpu.VMEM((2, CHUNK, D_EMB), jnp.int32),
            pltpu.VMEM((2, CHUNK, D_EMB // 2), jnp.int32),
            pltpu.SemaphoreType.DMA,
            pltpu.SemaphoreType.DMA,
            pltpu.SemaphoreType.DMA,
            pltpu.SemaphoreType.DMA,
        ],
    )
    def k(idx_hbm, table_hbm, out_hbm, idx_v, bufs_f, bufs_h,
          sem0, sem1, osem0, osem1):
        wid = lax.axis_index("s") * NC + lax.axis_index("c")
        base = wid * B_PER_W
        pltpu.sync_copy(idx_hbm.at[wid], idx_v)
        table_i = table_hbm.bitcast(jnp.int32)
        sems = [sem0, sem1]
        osems = [osem0, osem1]
        cps = [None, None]
        ocps = [None, None]
        cps[0] = pltpu.async_copy(
            table_i.at[idx_v.at[0]], bufs_f.at[0], sems[0])
        for c in range(NCHUNK):
            b = c % 2
            if c + 1 < NCHUNK:
                cps[1 - b] = pltpu.async_copy(
                    table_i.at[idx_v.at[c + 1]],
                    bufs_f.at[1 - b], sems[1 - b])
            cps[b].wait()
            if ocps[b] is not None:
                ocps[b].wait()

            @plsc.parallel_loop(0, CHUNK * GROUPS, unroll=8)
            def _conv(i, _b=b):
                r = i >> 4
                g = i & (GROUPS - 1)
                c0 = g * 32
                ua = bufs_f[_b, r, pl.ds(c0, 16)]
                ub = bufs_f[_b, r, pl.ds(c0 + 16, 16)]
                rnd = jnp.int32(0x8000)
                wa = lax.shift_right_logical(ua + rnd, jnp.int32(16))
                wb = (ub + rnd) & jnp.int32(-65536)
                bufs_h[_b, r, pl.ds(g * 16, 16)] = wa | wb

            ocps[b] = pltpu.async_copy(
                bufs_h.at[b],
                out_hbm.at[pl.ds(base + c * CHUNK, CHUNK)],
                osems[b])
        for b in range(2):
            if ocps[b] is not None:
                ocps[b].wait()

    return k(word_chunks, table)


def _tc_mlp_piece(x, Wlo, Whi, b1, out, piece):
    row0 = piece * N_PIECE

    def body(*refs):
        if len(refs) == 8:
            x_ref, wl_ref, wh_ref, b_ref, _o_in, o_ref, acc_vmem, sem = refs
        else:
            x_ref, wl_ref, wh_ref, b_ref, o_ref, acc_vmem, sem = refs
        j = pl.program_id(0)
        xw = x_ref[...]
        xl = lax.bitcast_convert_type(
            xw << jnp.int32(16), jnp.float32).astype(jnp.bfloat16)
        xh = lax.bitcast_convert_type(
            xw & jnp.int32(-65536), jnp.float32).astype(jnp.bfloat16)
        acc = jnp.dot(xl, wl_ref[...], preferred_element_type=jnp.float32)
        acc += jnp.dot(xh, wh_ref[...], preferred_element_type=jnp.float32)
        acc_vmem[...] = jnp.tanh(acc + b_ref[...])
        cp = pltpu.make_async_copy(
            acc_vmem, o_ref.at[pl.ds(row0 + j * BM, BM)], sem)
        cp.start()
        cp.wait()

    in_specs = [
        pl.BlockSpec((BM, D_EMB // 2), lambda i: (i, 0)),
        pl.BlockSpec((D_EMB // 2, S_DIM), lambda i: (0, 0)),
        pl.BlockSpec((D_EMB // 2, S_DIM), lambda i: (0, 0)),
        pl.BlockSpec((1, S_DIM), lambda i: (0, 0)),
    ]
    operands = [x, Wlo, Whi, b1.reshape(1, S_DIM)]
    aliases = {}
    if out is not None:
        in_specs.append(pl.BlockSpec(memory_space=pl.ANY))
        operands.append(out)
        aliases = {4: 0}

    return pl.pallas_call(
        body,
        grid=(N_PIECE // BM,),
        in_specs=in_specs,
        out_specs=pl.BlockSpec(memory_space=pl.ANY),
        out_shape=jax.ShapeDtypeStruct((N_TOK, S_DIM), jnp.float32),
        scratch_shapes=[
            pltpu.VMEM((BM, S_DIM), jnp.float32),
            pltpu.SemaphoreType.DMA,
        ],
        input_output_aliases=aliases,
    )(*operands)


def kernel(word, word_embeddings, grammar_preterminates, W1, b1):
    del grammar_preterminates
    word_chunks = word.astype(jnp.int32).reshape(P, NW, NCHUNK, CHUNK)
    w4 = W1.reshape(GROUPS, 2, 16, S_DIM)
    Wlo = w4[:, 0].reshape(D_EMB // 2, S_DIM).astype(jnp.bfloat16)
    Whi = w4[:, 1].reshape(D_EMB // 2, S_DIM).astype(jnp.bfloat16)
    gathered = [_sc_gather_piece(word_chunks[p], word_embeddings)
                for p in range(P)]
    out = None
    for p in range(P):
        out = _tc_mlp_piece(gathered[p], Wlo, Whi, b1, out, p)
    return out


# speedup vs baseline: 1.1510x; 1.1510x over previous
import functools

import jax
import jax.numpy as jnp
import numpy as np
from jax import lax
from jax.experimental import pallas as pl
from jax.experimental.pallas import tpu as pltpu
from jax.experimental.pallas import tpu_sc as plsc

N_TOK = 32768
D_EMB = 512
S_DIM = 256

NC = 2
NS = 16
NW = NC * NS

P = 2
N_PIECE = N_TOK // P
B_PER_W = N_PIECE // NW
CHUNK = 64
NCHUNK = B_PER_W // CHUNK
GROUPS = D_EMB // 32

BM = 1024

def _sc_gather_piece(word_chunks, table):
    mesh = plsc.VectorSubcoreMesh(core_axis_name="c", subcore_axis_name="s")

    @functools.partial(
        pl.kernel,
        mesh=mesh,
        out_type=jax.ShapeDtypeStruct((N_PIECE, D_EMB // 2), jnp.int32),
        scratch_types=[
            pltpu.VMEM((NCHUNK, CHUNK), jnp.int32),
            pltpu.VMEM((2, CHUNK, D_EMB), jnp.int32),
            pltpu.VMEM((2, CHUNK, D_EMB // 2), jnp.int32),
            pltpu.SemaphoreType.DMA,
            pltpu.SemaphoreType.DMA,
            pltpu.SemaphoreType.DMA,
            pltpu.SemaphoreType.DMA,
        ],
    )
    def k(idx_hbm, table_hbm, out_hbm, idx_v, bufs_f, bufs_h,
          sem0, sem1, osem0, osem1):
        wid = lax.axis_index("s") * NC + lax.axis_index("c")
        base = wid * B_PER_W
        pltpu.sync_copy(idx_hbm.at[wid], idx_v)
        table_i = table_hbm.bitcast(jnp.int32)
        sems = [sem0, sem1]
        osems = [osem0, osem1]
        cps = [None, None]
        ocps = [None, None]
        cps[0] = pltpu.async_copy(
            table_i.at[idx_v.at[0]], bufs_f.at[0], sems[0])
        for c in range(NCHUNK):
            b = c % 2
            if c + 1 < NCHUNK:
                cps[1 - b] = pltpu.async_copy(
                    table_i.at[idx_v.at[c + 1]],
                    bufs_f.at[1 - b], sems[1 - b])
            cps[b].wait()
            if ocps[b] is not None:
                ocps[b].wait()

            @plsc.parallel_loop(0, CHUNK * GROUPS, unroll=8)
            def _conv(i, _b=b):
                r = i >> 4
                g = i & (GROUPS - 1)
                c0 = g * 32
                ua = bufs_f[_b, r, pl.ds(c0, 16)]
                ub = bufs_f[_b, r, pl.ds(c0 + 16, 16)]
                rnd = jnp.int32(0x8000)
                wa = lax.shift_right_logical(ua + rnd, jnp.int32(16))
                wb = (ub + rnd) & jnp.int32(-65536)
                bufs_h[_b, r, pl.ds(g * 16, 16)] = wa | wb

            ocps[b] = pltpu.async_copy(
                bufs_h.at[b],
                out_hbm.at[pl.ds(base + c * CHUNK, CHUNK)],
                osems[b])
        for b in range(2):
            if ocps[b] is not None:
                ocps[b].wait()

    return k(word_chunks, table)


def _tc_mlp_piece(x, Wlo, Whi, b1, out, piece):
    row0 = piece * N_PIECE

    def body(*refs):
        if len(refs) == 8:
            x_ref, wl_ref, wh_ref, b_ref, _o_in, o_ref, acc_vmem, sem = refs
        else:
            x_ref, wl_ref, wh_ref, b_ref, o_ref, acc_vmem, sem = refs
        j = pl.program_id(0)
        xw = x_ref[...]
        xl = lax.bitcast_convert_type(
            xw << jnp.int32(16), jnp.float32).astype(jnp.bfloat16)
        xh = lax.bitcast_convert_type(
            xw & jnp.int32(-65536), jnp.float32).astype(jnp.bfloat16)
        acc = jnp.dot(xl, wl_ref[...], preferred_element_type=jnp.float32)
        acc += jnp.dot(xh, wh_ref[...], preferred_element_type=jnp.float32)
        acc_vmem[...] = jnp.tanh(acc + b_ref[...])
        cp = pltpu.make_async_copy(
            acc_vmem, o_ref.at[pl.ds(row0 + j * BM, BM)], sem)
        cp.start()
        cp.wait()

    in_specs = [
        pl.BlockSpec((BM, D_EMB // 2), lambda i: (i, 0)),
        pl.BlockSpec((D_EMB // 2, S_DIM), lambda i: (0, 0)),
        pl.BlockSpec((D_EMB // 2, S_DIM), lambda i: (0, 0)),
        pl.BlockSpec((1, S_DIM), lambda i: (0, 0)),
    ]
    operands = [x, Wlo, Whi, b1.reshape(1, S_DIM)]
    aliases = {}
    if out is not None:
        in_specs.append(pl.BlockSpec(memory_space=pl.ANY))
        operands.append(out)
        aliases = {4: 0}

    return pl.pallas_call(
        body,
        grid=(N_PIECE // BM,),
        in_specs=in_specs,
        out_specs=pl.BlockSpec(memory_space=pl.ANY),
        out_shape=jax.ShapeDtypeStruct((N_TOK, S_DIM), jnp.float32),
        scratch_shapes=[
            pltpu.VMEM((BM, S_DIM), jnp.float32),
            pltpu.SemaphoreType.DMA,
        ],
        input_output_aliases=aliases,
    )(*operands)


def kernel(word, word_embeddings, grammar_preterminates, W1, b1):
    del grammar_preterminates
    word_chunks = word.astype(jnp.int32).reshape(P, NW, NCHUNK, CHUNK)
    w4 = W1.reshape(GROUPS, 2, 16, S_DIM)
    Wlo = w4[:, 0].reshape(D_EMB // 2, S_DIM).astype(jnp.bfloat16)
    Whi = w4[:, 1].reshape(D_EMB // 2, S_DIM).astype(jnp.bfloat16)
    gathered = [_sc_gather_piece(word_chunks[p], word_embeddings)
                for p in range(P)]
    out = None
    for p in range(P):
        out = _tc_mlp_piece(gathered[p], Wlo, Whi, b1, out, p)
    return out
